# baseline (speedup 1.0000x reference)
import functools

import jax
import jax.numpy as jnp
from jax import lax
from jax.experimental import pallas as pl
from jax.experimental.pallas import tpu as pltpu
from jax.experimental.pallas import tpu_sc as plsc

VOCAB = 1_000_000
DIM = 64
BATCH = 16384
LANES = 16
NUM_CORES = 2
NUM_SUBCORES = 16
NUM_WORKERS = NUM_CORES * NUM_SUBCORES
B_PER_W = BATCH // NUM_WORKERS
CHUNK = 128
N_CHUNKS = B_PER_W // CHUNK

_LOG1P_COEFS = (
    5.629329962175689e-07,
    0.9999574422836304,
    -0.49920639395713806,
    0.3269723653793335,
    -0.2228347212076187,
    0.13076335191726685,
    -0.05262395367026329,
    0.01011890172958374,
)


def _softplus16(x):
    u = jnp.exp(-jnp.abs(x))
    acc = jnp.full((LANES,), _LOG1P_COEFS[-1], dtype=jnp.float32)
    for c in _LOG1P_COEFS[-2::-1]:
        acc = acc * u + jnp.float32(c)
    return jnp.maximum(x, jnp.float32(0.0)) + acc


def _sc_body(idx_hbm, table_hbm, out_hbm, idx_v, g_v, buf_v, outst_v, sem):
    wid = lax.axis_index("s") * NUM_CORES + lax.axis_index("c")
    base = wid * B_PER_W
    pltpu.sync_copy(idx_hbm.at[pl.ds(base, B_PER_W)], idx_v)

    for t in range(B_PER_W // LANES):
        sl = pl.ds(t * LANES, LANES)
        g_v[sl] = idx_v[sl] >> 1

    def chunk_body(c, carry):
        pltpu.async_copy(
            table_hbm.at[g_v.at[pl.ds(c * CHUNK, CHUNK)]], buf_v, sem
        ).wait()

        def row_body(r, carry2):
            rsplat = jnp.zeros((LANES,), jnp.int32) + (c * CHUNK + r)
            iv = plsc.load_gather(idx_v, [rsplat])
            hf = jnp.bitwise_and(iv, jnp.int32(1)).astype(jnp.float32)
            for t in range(DIM // LANES):
                x0 = buf_v[r, pl.ds(t * LANES, LANES)]
                x1 = buf_v[r, pl.ds(DIM + t * LANES, LANES)]
                x = x0 + hf * (x1 - x0)
                outst_v[r, pl.ds(t * LANES, LANES)] = _softplus16(x)
            return carry2

        lax.fori_loop(0, CHUNK, row_body, 0)
        pltpu.sync_copy(outst_v, out_hbm.at[pl.ds(base + c * CHUNK, CHUNK)])
        return carry

    lax.fori_loop(0, N_CHUNKS, chunk_body, 0)


def kernel(target_w_id, S):
    from jax.experimental import layout as jexp_layout

    idx = target_w_id.astype(jnp.int32)
    table2 = jexp_layout.with_layout_constraint(
        S.reshape(VOCAB // 2, 2 * DIM), jexp_layout.Layout((1, 0))
    )
    mesh = plsc.VectorSubcoreMesh(core_axis_name="c", subcore_axis_name="s")
    run = pl.kernel(
        _sc_body,
        mesh=mesh,
        out_type=jax.ShapeDtypeStruct((BATCH, DIM), jnp.float32),
        scratch_types=[
            pltpu.VMEM((B_PER_W,), jnp.int32),
            pltpu.VMEM((B_PER_W,), jnp.int32),
            pltpu.VMEM((CHUNK, 2 * DIM), jnp.float32),
            pltpu.VMEM((CHUNK, DIM), jnp.float32),
            pltpu.SemaphoreType.DMA,
        ],
        compiler_params=pltpu.CompilerParams(
            needs_layout_passes=False, skip_device_barrier=True
        ),
    )
    return run(idx, table2)


# speedup vs baseline: 1.8152x; 1.8152x over previous
import functools

import jax
import jax.numpy as jnp
from jax import lax
from jax.experimental import pallas as pl
from jax.experimental.pallas import tpu as pltpu
from jax.experimental.pallas import tpu_sc as plsc

VOCAB = 1_000_000
DIM = 64
BATCH = 16384
LANES = 16
NUM_CORES = 2
NUM_SUBCORES = 16
NUM_WORKERS = NUM_CORES * NUM_SUBCORES
B_PER_W = BATCH // NUM_WORKERS
NBUF = 8
OUT_CHUNK = 128
TAIL_START = (VOCAB // 128) * 128
LAST_SLAB = TAIL_START // 128 - 1

_LOG1P_COEFS = (
    5.629329962175689e-07,
    0.9999574422836304,
    -0.49920639395713806,
    0.3269723653793335,
    -0.2228347212076187,
    0.13076335191726685,
    -0.05262395367026329,
    0.01011890172958374,
)


def _softplus16(x):
    u = jnp.exp(-jnp.abs(x))
    acc = jnp.full((LANES,), _LOG1P_COEFS[-1], dtype=jnp.float32)
    for c in _LOG1P_COEFS[-2::-1]:
        acc = acc * u + jnp.float32(c)
    return jnp.maximum(x, jnp.float32(0.0)) + acc


def _sc_body(idx_hbm, table_hbm, tail_hbm, out_hbm, idx_v, tail_v, outst_v,
             *blk_and_sem):
    blks = blk_and_sem[:NBUF]
    sems = blk_and_sem[NBUF:]
    wid = lax.axis_index("s") * NUM_CORES + lax.axis_index("c")
    base = wid * B_PER_W
    pltpu.sync_copy(idx_hbm.at[pl.ds(base, B_PER_W)], idx_v)
    pltpu.sync_copy(tail_hbm, tail_v)

    iota = lax.iota(jnp.int32, LANES)
    jg_t = [(t * LANES + iota) >> 3 for t in range(DIM // LANES)]
    jl_t = [jnp.bitwise_and(t * LANES + iota, jnp.int32(7))
            for t in range(DIM // LANES)]

    def splat_idx(b):
        iv = plsc.load_gather(idx_v, [jnp.zeros((LANES,), jnp.int32) + b])
        return jnp.max(iv), iv

    def issue(k, b):
        i_s, _ = splat_idx(b)
        slab = jnp.minimum(i_s >> 7, jnp.int32(LAST_SLAB))
        off = pl.multiple_of(slab * 128, 128)
        pltpu.async_copy(table_hbm.at[:, :, pl.ds(off, 128)], blks[k], sems[k])

    def group_body(g, carry):
        for k in range(NBUF):
            issue(k, g * NBUF + k)
        for k in range(NBUF):
            pltpu.make_async_copy(
                table_hbm.at[:, :, pl.ds(0, 128)], blks[k], sems[k]
            ).wait()
        for k in range(NBUF):
            b = g * NBUF + k
            i_s, iv = splat_idx(b)
            slab = jnp.minimum(i_s >> 7, jnp.int32(LAST_SLAB))
            ilv = jnp.minimum(iv - slab * 128, jnp.int32(127))
            is_tail = iv >= jnp.int32(TAIL_START)
            rt = jnp.clip(i_s - jnp.int32(TAIL_START), 0, DIM - 1)
            bloc = jnp.bitwise_and(b, jnp.int32(OUT_CHUNK - 1))
            for t in range(DIM // LANES):
                xn = plsc.load_gather(blks[k], [jg_t[t], jl_t[t], ilv])
                xt = tail_v[rt, pl.ds(t * LANES, LANES)]
                x = jnp.where(is_tail, xt, xn)
                outst_v[bloc, pl.ds(t * LANES, LANES)] = _softplus16(x)

        @pl.when(jnp.bitwise_and(g, jnp.int32(15)) == jnp.int32(15))
        def _flush():
            chunk = g >> 4
            pltpu.sync_copy(
                outst_v, out_hbm.at[pl.ds(base + chunk * OUT_CHUNK, OUT_CHUNK)]
            )

        return carry

    lax.fori_loop(0, B_PER_W // NBUF, group_body, 0)


def kernel(target_w_id, S):
    idx = target_w_id.astype(jnp.int32)
    table3 = S.T.reshape(8, 8, VOCAB)
    tail = S[TAIL_START:, :]
    mesh = plsc.VectorSubcoreMesh(core_axis_name="c", subcore_axis_name="s")
    run = pl.kernel(
        _sc_body,
        mesh=mesh,
        out_type=jax.ShapeDtypeStruct((BATCH, DIM), jnp.float32),
        scratch_types=(
            [
                pltpu.VMEM((B_PER_W,), jnp.int32),
                pltpu.VMEM((VOCAB - TAIL_START, DIM), jnp.float32),
                pltpu.VMEM((OUT_CHUNK, DIM), jnp.float32),
            ]
            + [pltpu.VMEM((8, 8, 128), jnp.float32) for _ in range(NBUF)]
            + [pltpu.SemaphoreType.DMA for _ in range(NBUF)]
        ),
        compiler_params=pltpu.CompilerParams(needs_layout_passes=False),
    )
    return run(idx, table3, tail)


# speedup vs baseline: 2.1600x; 1.1899x over previous
import functools

import jax
import jax.numpy as jnp
from jax import lax
from jax.experimental import pallas as pl
from jax.experimental.pallas import tpu as pltpu
from jax.experimental.pallas import tpu_sc as plsc

VOCAB = 1_000_000
DIM = 64
BATCH = 16384
LANES = 16
NUM_CORES = 2
NUM_SUBCORES = 16
NUM_WORKERS = NUM_CORES * NUM_SUBCORES
B_PER_W = BATCH // NUM_WORKERS
NBUF = 8
OUT_CHUNK = 128
TAIL_START = (VOCAB // 128) * 128
LAST_SLAB = TAIL_START // 128 - 1

_LOG1P_COEFS = (
    5.629329962175689e-07,
    0.9999574422836304,
    -0.49920639395713806,
    0.3269723653793335,
    -0.2228347212076187,
    0.13076335191726685,
    -0.05262395367026329,
    0.01011890172958374,
)


def _softplus16(x):
    u = jnp.exp(-jnp.abs(x))
    acc = jnp.full((LANES,), _LOG1P_COEFS[-1], dtype=jnp.float32)
    for c in _LOG1P_COEFS[-2::-1]:
        acc = acc * u + jnp.float32(c)
    return jnp.maximum(x, jnp.float32(0.0)) + acc


def _sc_body(idx_hbm, table_hbm, tail_hbm, out_hbm, idx_v, tail_v, outst_v,
             *blk_and_sem):
    blks = blk_and_sem[:NBUF]
    sems = blk_and_sem[NBUF:]
    wid = lax.axis_index("s") * NUM_CORES + lax.axis_index("c")
    base = wid * B_PER_W
    pltpu.sync_copy(idx_hbm.at[pl.ds(base, B_PER_W)], idx_v)
    pltpu.sync_copy(tail_hbm, tail_v)

    iota = lax.iota(jnp.int32, LANES)
    jg_t = [(t * LANES + iota) >> 3 for t in range(DIM // LANES)]
    jl_t = [jnp.bitwise_and(t * LANES + iota, jnp.int32(7))
            for t in range(DIM // LANES)]

    def splat_idx(b):
        iv = plsc.load_gather(idx_v, [jnp.zeros((LANES,), jnp.int32) + b])
        return jnp.max(iv), iv

    def issue(k, b):
        i_s, _ = splat_idx(b)
        slab = jnp.minimum(i_s >> 7, jnp.int32(LAST_SLAB))
        off = pl.multiple_of(slab * 128, 128)
        pltpu.async_copy(table_hbm.at[:, :, pl.ds(off, 128)], blks[k], sems[k])

    def extract(k, b):
        i_s, iv = splat_idx(b)
        slab = jnp.minimum(i_s >> 7, jnp.int32(LAST_SLAB))
        ilv = jnp.minimum(iv - slab * 128, jnp.int32(127))
        is_tail = iv >= jnp.int32(TAIL_START)
        rt = jnp.clip(i_s - jnp.int32(TAIL_START), 0, DIM - 1)
        bloc = jnp.bitwise_and(b, jnp.int32(OUT_CHUNK - 1))
        for t in range(DIM // LANES):
            xn = plsc.load_gather(blks[k], [jg_t[t], jl_t[t], ilv])
            xt = tail_v[rt, pl.ds(t * LANES, LANES)]
            x = jnp.where(is_tail, xt, xn)
            outst_v[bloc, pl.ds(t * LANES, LANES)] = _softplus16(x)

    def group_body(g, carry):
        for k in range(NBUF):
            issue(k, g * NBUF + k)
        for k in range(NBUF):
            pltpu.make_async_copy(
                table_hbm.at[:, :, pl.ds(0, 128)], blks[k], sems[k]
            ).wait()
            extract(k, g * NBUF + k)

        @pl.when(jnp.bitwise_and(g, jnp.int32(15)) == jnp.int32(15))
        def _flush():
            chunk = g >> 4
            pltpu.sync_copy(
                outst_v, out_hbm.at[pl.ds(base + chunk * OUT_CHUNK, OUT_CHUNK)]
            )

        return carry

    lax.fori_loop(0, B_PER_W // NBUF, group_body, 0)


def kernel(target_w_id, S):
    idx = target_w_id.astype(jnp.int32)
    table3 = S.T.reshape(8, 8, VOCAB)
    tail = S[TAIL_START:, :]
    mesh = plsc.VectorSubcoreMesh(core_axis_name="c", subcore_axis_name="s")
    run = pl.kernel(
        _sc_body,
        mesh=mesh,
        out_type=jax.ShapeDtypeStruct((BATCH, DIM), jnp.float32),
        scratch_types=(
            [
                pltpu.VMEM((B_PER_W,), jnp.int32),
                pltpu.VMEM((VOCAB - TAIL_START, DIM), jnp.float32),
                pltpu.VMEM((OUT_CHUNK, DIM), jnp.float32),
            ]
            + [pltpu.VMEM((8, 8, 128), jnp.float32) for _ in range(NBUF)]
            + [pltpu.SemaphoreType.DMA for _ in range(NBUF)]
        ),
        compiler_params=pltpu.CompilerParams(needs_layout_passes=False),
    )
    return run(idx, table3, tail)
